# baseline (speedup 1.0000x reference)
import functools

import jax
import jax.numpy as jnp
import numpy as np
from jax.experimental import pallas as pl
from jax.experimental.pallas import tpu as pltpu


def _sel(ndj, nw, nj):
    t = np.zeros((ndj, nw, nj), np.float32)
    for dj in range(ndj):
        for j in range(nj):
            w = 2 * j - 1 + dj
            if 0 <= w < nw:
                t[dj, w, j] = 1.0
    return t


_T1 = _sel(3, 32, 16)
_T2 = _sel(3, 16, 8)


def _selectors():
    g1 = np.zeros((384, 576), np.float32)
    for di in range(3):
        for c in range(4):
            for w in range(32):
                for j in range(16):
                    for dj in range(3):
                        if _T1[dj, w, j]:
                            g1[di * 128 + c * 32 + w,
                               j * 36 + c * 9 + di * 3 + dj] = 1.0
    g2 = np.zeros((768, 1152), np.float32)
    for di in range(3):
        for a in range(16):
            for c1 in range(16):
                for b in range(8):
                    for dj in range(3):
                        if _T2[dj, a, b]:
                            g2[di * 256 + a * 16 + c1,
                               b * 144 + c1 * 9 + di * 3 + dj] = 1.0
    return g1, g2


_G1X, _G2X = _selectors()
_P2 = np.zeros((256, 256), np.float32)
for _b in range(8):
    for _o in range(32):
        _P2[_b * 32 + _o, _o * 8 + _b] = 1.0


def _fused_kernel(bt, x_ref, w1_ref, w2_ref, g1_ref, g2_ref, p2_ref, b1_ref,
                  b2_ref, fc_ref, rg_ref, fcb_ref, lw_ref, lb_ref, rgb_ref,
                  out_last_ref, out_reg_ref, w1x, w2x, amat_s, bmat_s, fcp):
    bf16 = jnp.bfloat16

    @pl.when(pl.program_id(0) == 0)
    def _build():
        tb = (((1,), (1,)), ((), ()))
        w1x[...] = jnp.zeros_like(w1x)
        w2x[...] = jnp.zeros_like(w2x)
        w1b = w1_ref[...].astype(bf16)
        for j in range(16):
            w1x[j * 16:(j + 1) * 16, j * 36:(j + 1) * 36] = w1b
        w2b = w2_ref[...].astype(bf16)
        for b in range(8):
            w2x[b * 32:(b + 1) * 32, b * 144:(b + 1) * 144] = w2b
        amat_s[...] = jax.lax.dot_general(
            g1_ref[...], w1x[...], tb,
            preferred_element_type=jnp.float32).astype(bf16)
        bm = jax.lax.dot_general(
            g2_ref[...], w2x[...], tb,
            preferred_element_type=jnp.float32).astype(bf16)
        bmat_s[...] = jnp.dot(
            bm, p2_ref[...], preferred_element_type=jnp.float32).astype(bf16)
        for i2 in range(8):
            for c2 in range(32):
                s = c2 * 64 + i2 * 8
                d = i2 * 256 + c2 * 8
                fcp[d:d + 8, :256] = fc_ref[s:s + 8, :].astype(bf16)
                fcp[d:d + 8, 256:] = rg_ref[s:s + 8, :].astype(bf16)
    _body(bt, x_ref, amat_s, bmat_s, b1_ref, b2_ref, fcp, fcb_ref,
          lw_ref, lb_ref, rgb_ref, out_last_ref, out_reg_ref)


def _body(bt, x_ref, a_ref, b_ref, b1_ref, b2_ref, fc_ref, fcb_ref,
          lw_ref, lb_ref, rgb_ref, out_last_ref, out_reg_ref):
    f32 = jnp.float32
    bf16 = jnp.bfloat16
    dot = functools.partial(jnp.dot, preferred_element_type=f32)

    xb = x_ref[...].astype(bf16)
    xp = []
    for p in range(4):
        rows = []
        for i2 in range(8):
            h = 4 * i2 + p
            rows.append(jnp.concatenate(
                [xb[:, c * 1024 + h * 32: c * 1024 + h * 32 + 32]
                 for c in range(4)], axis=1))
        xp.append(jnp.concatenate(rows, axis=0))
    x0, x1, x2, x3 = xp

    a0, a1, a2 = a_ref[:128], a_ref[128:256], a_ref[256:]
    b_ref = [b_ref[:256], b_ref[256:512], b_ref[512:]]
    b1 = jnp.tile(b1_ref[...], (1, 16))

    zx = jnp.zeros((bt, 128), bf16)
    x3s = jnp.concatenate([zx, x3[: 7 * bt]], axis=0)
    h_e = jnp.maximum(dot(x3s, a0) + dot(x0, a1) + dot(x1, a2) + b1, 0.0)
    h_o = jnp.maximum(dot(x1, a0) + dot(x2, a1) + dot(x3, a2) + b1, 0.0)
    h_e = h_e.astype(bf16)
    h_o = h_o.astype(bf16)

    zh = jnp.zeros((bt, 256), bf16)
    h_os = jnp.concatenate([zh, h_o[: 7 * bt]], axis=0)
    out2 = jnp.maximum(
        dot(h_os, b_ref[0]) + dot(h_e, b_ref[1]) + dot(h_o, b_ref[2])
        + jnp.repeat(b2_ref[...], 8, axis=1), 0.0).astype(bf16)

    hr = dot(out2[:bt], fc_ref[:256])
    for i2 in range(1, 8):
        hr += dot(out2[i2 * bt:(i2 + 1) * bt],
                  fc_ref[i2 * 256:(i2 + 1) * 256])

    h = jnp.maximum(hr[:, :256] + fcb_ref[...], 0.0).astype(bf16)
    out_last_ref[...] = dot(h, lw_ref[...]) + lb_ref[...]
    out_reg_ref[...] = hr[:, 256:] + rgb_ref[...]


def kernel(x_flat, conv0_w, conv0_b, conv1_w, conv1_b, fc0_w, fc0_b,
           last_w, last_b, reg_w, reg_b):
    f32 = jnp.float32
    bf16 = jnp.bfloat16
    B = x_flat.shape[0]
    bt = 256 if B % 256 == 0 else B


    full = lambda a: pl.BlockSpec(a.shape, lambda i: (0,) * a.ndim)
    weights = [conv0_w.reshape(16, 36), conv1_w.reshape(32, 144),
               jnp.asarray(_G1X, bf16), jnp.asarray(_G2X, bf16),
               jnp.asarray(_P2, bf16),
               conv0_b.reshape(1, 16), conv1_b.reshape(1, 32),
               fc0_w, reg_w, fc0_b.reshape(1, 256).astype(f32),
               last_w.astype(bf16), last_b.reshape(1, 128).astype(f32),
               reg_b.reshape(1, 64).astype(f32)]

    out_last, out_reg = pl.pallas_call(
        functools.partial(_fused_kernel, bt),
        out_shape=(jax.ShapeDtypeStruct((B, 128), f32),
                   jax.ShapeDtypeStruct((B, 64), f32)),
        grid=(B // bt,),
        in_specs=[pl.BlockSpec((bt, 4096), lambda i: (i, 0))]
        + [full(w) for w in weights],
        out_specs=[pl.BlockSpec((bt, 128), lambda i: (i, 0)),
                   pl.BlockSpec((bt, 64), lambda i: (i, 0))],
        scratch_shapes=[pltpu.VMEM((256, 576), jnp.bfloat16),
                        pltpu.VMEM((256, 1152), jnp.bfloat16),
                        pltpu.VMEM((384, 256), jnp.bfloat16),
                        pltpu.VMEM((768, 256), jnp.bfloat16),
                        pltpu.VMEM((2048, 320), jnp.bfloat16)],
        compiler_params=pltpu.CompilerParams(
            dimension_semantics=("arbitrary",)),
    )(x_flat, *weights)
    return out_last, out_reg


# speedup vs baseline: 1.0406x; 1.0406x over previous
import functools

import jax
import jax.numpy as jnp
import numpy as np
from jax.experimental import pallas as pl
from jax.experimental.pallas import tpu as pltpu


def _sel(ndj, nw, nj):
    t = np.zeros((ndj, nw, nj), np.float32)
    for dj in range(ndj):
        for j in range(nj):
            w = 2 * j - 1 + dj
            if 0 <= w < nw:
                t[dj, w, j] = 1.0
    return t


_T1 = _sel(3, 32, 16)
_T2 = _sel(3, 16, 8)


def _selectors():
    g1 = np.zeros((384, 576), np.float32)
    for di in range(3):
        for c in range(4):
            for w in range(32):
                for j in range(16):
                    for dj in range(3):
                        if _T1[dj, w, j]:
                            g1[di * 128 + c * 32 + w,
                               j * 36 + c * 9 + di * 3 + dj] = 1.0
    g2 = np.zeros((768, 1152), np.float32)
    for di in range(3):
        for a in range(16):
            for c1 in range(16):
                for b in range(8):
                    for dj in range(3):
                        if _T2[dj, a, b]:
                            g2[di * 256 + a * 16 + c1,
                               b * 144 + c1 * 9 + di * 3 + dj] = 1.0
    return g1, g2


_G1X, _G2X = _selectors()


def _fused_kernel(bt, x_ref, w1_ref, w2_ref, g1_ref, g2_ref, b1_ref, b2_ref,
                  fc_ref, fcb_ref, lw_ref, lb_ref, rgb_ref,
                  out_last_ref, out_reg_ref, w1x, w2x, amat_s, bmat_s):
    bf16 = jnp.bfloat16

    @pl.when(pl.program_id(0) == 0)
    def _build():
        tb = (((1,), (1,)), ((), ()))
        w1x[...] = jnp.zeros_like(w1x)
        w2x[...] = jnp.zeros_like(w2x)
        w1b = w1_ref[...].astype(bf16)
        for j in range(16):
            w1x[j * 16:(j + 1) * 16, j * 36:(j + 1) * 36] = w1b
        w2b = w2_ref[...].astype(bf16)
        for b in range(8):
            w2x[b * 32:(b + 1) * 32, b * 144:(b + 1) * 144] = w2b
        amat_s[...] = jax.lax.dot_general(
            g1_ref[...], w1x[...], tb,
            preferred_element_type=jnp.float32).astype(bf16)
        bmat_s[...] = jax.lax.dot_general(
            g2_ref[...], w2x[...], tb,
            preferred_element_type=jnp.float32).astype(bf16)
    _body(bt, x_ref, amat_s, bmat_s, b1_ref, b2_ref, fc_ref, fcb_ref,
          lw_ref, lb_ref, rgb_ref, out_last_ref, out_reg_ref)


def _body(bt, x_ref, a_ref, b_ref, b1_ref, b2_ref, fc_ref, fcb_ref,
          lw_ref, lb_ref, rgb_ref, out_last_ref, out_reg_ref):
    f32 = jnp.float32
    bf16 = jnp.bfloat16
    dot = functools.partial(jnp.dot, preferred_element_type=f32)

    xb = x_ref[...].astype(bf16)
    xp = []
    for p in range(4):
        rows = []
        for i2 in range(8):
            h = 4 * i2 + p
            rows.append(jnp.concatenate(
                [xb[:, c * 1024 + h * 32: c * 1024 + h * 32 + 32]
                 for c in range(4)], axis=1))
        xp.append(jnp.concatenate(rows, axis=0))
    x0, x1, x2, x3 = xp

    a0, a1, a2 = a_ref[:128], a_ref[128:256], a_ref[256:]
    b_ref = [b_ref[:256], b_ref[256:512], b_ref[512:]]
    b1 = jnp.tile(b1_ref[...], (1, 16))

    zx = jnp.zeros((bt, 128), bf16)
    x3s = jnp.concatenate([zx, x3[: 7 * bt]], axis=0)
    h_e = jnp.maximum(dot(x3s, a0) + dot(x0, a1) + dot(x1, a2) + b1, 0.0)
    h_o = jnp.maximum(dot(x1, a0) + dot(x2, a1) + dot(x3, a2) + b1, 0.0)
    h_e = h_e.astype(bf16)
    h_o = h_o.astype(bf16)

    zh = jnp.zeros((bt, 256), bf16)
    h_os = jnp.concatenate([zh, h_o[: 7 * bt]], axis=0)
    out2 = jnp.maximum(
        dot(h_os, b_ref[0]) + dot(h_e, b_ref[1]) + dot(h_o, b_ref[2])
        + jnp.tile(b2_ref[...], (1, 8)), 0.0).astype(bf16)

    hr = dot(out2[:bt], fc_ref[0])
    for i2 in range(1, 8):
        hr += dot(out2[i2 * bt:(i2 + 1) * bt], fc_ref[i2])

    h = jnp.maximum(hr[:, :256] + fcb_ref[...], 0.0).astype(bf16)
    out_last_ref[...] = dot(h, lw_ref[...]) + lb_ref[...]
    out_reg_ref[...] = hr[:, 256:] + rgb_ref[...]


def kernel(x_flat, conv0_w, conv0_b, conv1_w, conv1_b, fc0_w, fc0_b,
           last_w, last_b, reg_w, reg_b):
    f32 = jnp.float32
    bf16 = jnp.bfloat16
    B = x_flat.shape[0]
    bt = 256 if B % 256 == 0 else B


    fcrg = jnp.concatenate(
        [fc0_w.reshape(32, 8, 8, 256), reg_w.reshape(32, 8, 8, 64)],
        axis=3).transpose(1, 2, 0, 3).reshape(8, 256, 320).astype(bf16)

    full = lambda a: pl.BlockSpec(a.shape, lambda i: (0,) * a.ndim)
    weights = [conv0_w.reshape(16, 36), conv1_w.reshape(32, 144),
               jnp.asarray(_G1X, bf16), jnp.asarray(_G2X, bf16),
               conv0_b.reshape(1, 16), conv1_b.reshape(1, 32),
               fcrg, fc0_b.reshape(1, 256).astype(f32),
               last_w.astype(bf16), last_b.reshape(1, 128).astype(f32),
               reg_b.reshape(1, 64).astype(f32)]

    out_last, out_reg = pl.pallas_call(
        functools.partial(_fused_kernel, bt),
        out_shape=(jax.ShapeDtypeStruct((B, 128), f32),
                   jax.ShapeDtypeStruct((B, 64), f32)),
        grid=(B // bt,),
        in_specs=[pl.BlockSpec((bt, 4096), lambda i: (i, 0))]
        + [full(w) for w in weights],
        out_specs=[pl.BlockSpec((bt, 128), lambda i: (i, 0)),
                   pl.BlockSpec((bt, 64), lambda i: (i, 0))],
        scratch_shapes=[pltpu.VMEM((256, 576), jnp.bfloat16),
                        pltpu.VMEM((256, 1152), jnp.bfloat16),
                        pltpu.VMEM((384, 256), jnp.bfloat16),
                        pltpu.VMEM((768, 256), jnp.bfloat16)],
        compiler_params=pltpu.CompilerParams(
            dimension_semantics=("arbitrary",)),
    )(x_flat, *weights)
    return out_last, out_reg
